# Initial kernel scaffold; baseline (speedup 1.0000x reference)
import jax
import jax.numpy as jnp
from jax.experimental import pallas as pl


def kernel(feat, edge_index, learnable_diag, W, b):
    raise NotImplementedError("write your pallas kernel here")


# speedup vs baseline: 2.9927x; 2.9927x over previous
import functools

import jax
import jax.numpy as jnp
from jax import lax
from jax.experimental import pallas as pl
from jax.experimental.pallas import tpu as pltpu
from jax.experimental.pallas import tpu_sc as plsc

N = 10000
E = 320000
D = 128
NC, NS = 2, 16
NW = NC * NS
C = 128
CH = 80
EPW = CH * C
EPAD = NW * EPW
DUMP = N
NPAD = 10240
STR = NPAD // NS
GRID = 10
BR = N // GRID

_mesh = plsc.VectorSubcoreMesh(core_axis_name="c", subcore_axis_name="s")


@functools.partial(
    pl.kernel,
    out_type=jax.ShapeDtypeStruct((NC * NPAD, D), jnp.float32),
    mesh=_mesh,
    scratch_types=[
        pltpu.VMEM((CH, C), jnp.int32),
        pltpu.VMEM((C, D), jnp.float32),
        pltpu.VMEM_SHARED((NPAD, D), jnp.float32),
    ],
)
def _sc_degree(dst_hbm, out_hbm, idx_v, rows_v, acc_sh):
    c = lax.axis_index("c")
    s = lax.axis_index("s")
    wid = s * NC + c
    pltpu.sync_copy(dst_hbm.at[wid], idx_v)
    zrow = jnp.zeros((16,), jnp.float32)

    def zinit(i, carry):
        for k in range(D // 16):
            rows_v[i, pl.ds(k * 16, 16)] = zrow
        return carry

    lax.fori_loop(0, C, zinit, 0)
    for k in range(STR // C):
        pltpu.sync_copy(rows_v, acc_sh.at[pl.ds(s * STR + k * C, C)])
    plsc.subcore_barrier()
    orow = jnp.ones((16,), jnp.float32)

    def oinit(i, carry):
        for k in range(D // 16):
            rows_v[i, pl.ds(k * 16, 16)] = orow
        return carry

    lax.fori_loop(0, C, oinit, 0)

    def body(j, carry):
        pltpu.sync_copy(rows_v, acc_sh.at[idx_v.at[j]], add=True)
        return carry

    lax.fori_loop(0, CH, body, 0)
    plsc.subcore_barrier()
    for k in range(STR // C):
        pltpu.sync_copy(acc_sh.at[pl.ds(s * STR + k * C, C)], rows_v)
        pltpu.sync_copy(rows_v, out_hbm.at[pl.ds(c * NPAD + s * STR + k * C, C)])


@functools.partial(
    pl.kernel,
    out_type=jax.ShapeDtypeStruct((NC * NPAD, D), jnp.float32),
    mesh=_mesh,
    scratch_types=[
        pltpu.VMEM((CH, C), jnp.int32),
        pltpu.VMEM((CH, C), jnp.int32),
        pltpu.VMEM((C, D), jnp.float32),
        pltpu.VMEM_SHARED((NPAD, D), jnp.float32),
        pltpu.SemaphoreType.DMA,
    ],
)
def _sc_edge_pass(m_hbm, src_hbm, dst_hbm, out_hbm,
                  sidx_v, didx_v, rows_v, acc_sh, sem):
    c = lax.axis_index("c")
    s = lax.axis_index("s")
    wid = s * NC + c
    pltpu.sync_copy(src_hbm.at[wid], sidx_v)
    pltpu.sync_copy(dst_hbm.at[wid], didx_v)
    zrow = jnp.zeros((16,), jnp.float32)

    def zinit(i, carry):
        for k in range(D // 16):
            rows_v[i, pl.ds(k * 16, 16)] = zrow
        return carry

    lax.fori_loop(0, C, zinit, 0)
    for k in range(STR // C):
        pltpu.sync_copy(rows_v, acc_sh.at[pl.ds(s * STR + k * C, C)])
    plsc.subcore_barrier()

    def body(j, carry):
        pltpu.async_copy(m_hbm.at[sidx_v.at[j]], rows_v, sem).wait()
        pltpu.sync_copy(rows_v, acc_sh.at[didx_v.at[j]], add=True)
        return carry

    lax.fori_loop(0, CH, body, 0)
    plsc.subcore_barrier()
    for k in range(STR // C):
        pltpu.sync_copy(acc_sh.at[pl.ds(s * STR + k * C, C)], rows_v)
        pltpu.sync_copy(rows_v, out_hbm.at[pl.ds(c * NPAD + s * STR + k * C, C)])


def _tc_dinv_m1_body(parts_ref, feat_ref, dinv_ref, m1_ref):
    deg = parts_ref[0, :, 0:1] + parts_ref[1, :, 0:1]
    dinv = lax.rsqrt(jnp.maximum(deg, 1.0))
    dinv_ref[...] = dinv
    m1_ref[...] = feat_ref[...] * dinv


_tc_dinv_m1 = pl.pallas_call(
    _tc_dinv_m1_body,
    grid=(GRID,),
    in_specs=[
        pl.BlockSpec((NC, BR, D), lambda i: (0, i, 0)),
        pl.BlockSpec((BR, D), lambda i: (i, 0)),
    ],
    out_specs=[
        pl.BlockSpec((BR, 1), lambda i: (i, 0)),
        pl.BlockSpec((BR, D), lambda i: (i, 0)),
    ],
    out_shape=[
        jax.ShapeDtypeStruct((N, 1), jnp.float32),
        jax.ShapeDtypeStruct((N, D), jnp.float32),
    ],
)


def _tc_f1_m2_body(feat_ref, dinv_ref, parts_ref, ld_ref, f1_ref, m2_ref):
    agg = parts_ref[0] + parts_ref[1]
    dinv = dinv_ref[...]
    f1 = feat_ref[...] - (agg * dinv) * ld_ref[1:2, :]
    f1_ref[...] = f1
    m2_ref[...] = f1 * dinv


_tc_f1_m2 = pl.pallas_call(
    _tc_f1_m2_body,
    grid=(GRID,),
    in_specs=[
        pl.BlockSpec((BR, D), lambda i: (i, 0)),
        pl.BlockSpec((BR, 1), lambda i: (i, 0)),
        pl.BlockSpec((NC, BR, D), lambda i: (0, i, 0)),
        pl.BlockSpec((3, D), lambda i: (0, 0)),
    ],
    out_specs=[
        pl.BlockSpec((BR, D), lambda i: (i, 0)),
        pl.BlockSpec((BR, D), lambda i: (i, 0)),
    ],
    out_shape=[
        jax.ShapeDtypeStruct((N, D), jnp.float32),
        jax.ShapeDtypeStruct((N, D), jnp.float32),
    ],
)


def _tc_final_body(feat_ref, dinv_ref, f1_ref, parts_ref, ld_ref, w_ref, b_ref,
                   h_ref):
    agg2 = parts_ref[0] + parts_ref[1]
    dinv = dinv_ref[...]
    f1 = f1_ref[...]
    f2 = f1 - (agg2 * dinv) * ld_ref[2:3, :]
    xm = (0.5 * feat_ref[...]) * ld_ref[0:1, :]
    h_lin = lax.dot_general(
        xm, w_ref[...], (((1,), (1,)), ((), ())),
        preferred_element_type=jnp.float32,
        precision=lax.Precision.HIGHEST,
    )
    h_ref[...] = h_lin + b_ref[...] + 0.5 * f1 + 0.5 * f2


_tc_final = pl.pallas_call(
    _tc_final_body,
    grid=(GRID,),
    in_specs=[
        pl.BlockSpec((BR, D), lambda i: (i, 0)),
        pl.BlockSpec((BR, 1), lambda i: (i, 0)),
        pl.BlockSpec((BR, D), lambda i: (i, 0)),
        pl.BlockSpec((NC, BR, D), lambda i: (0, i, 0)),
        pl.BlockSpec((3, D), lambda i: (0, 0)),
        pl.BlockSpec((D, D), lambda i: (0, 0)),
        pl.BlockSpec((1, D), lambda i: (0, 0)),
    ],
    out_specs=pl.BlockSpec((BR, D), lambda i: (i, 0)),
    out_shape=jax.ShapeDtypeStruct((N, D), jnp.float32),
)


def kernel(feat, edge_index, learnable_diag, W, b):
    src = edge_index[0].astype(jnp.int32)
    dst = edge_index[1].astype(jnp.int32)
    pad = EPAD - E
    src_p = jnp.concatenate([src, jnp.zeros((pad,), jnp.int32)]).reshape(NW, CH, C)
    dst_p = jnp.concatenate([dst, jnp.full((pad,), DUMP, jnp.int32)]).reshape(NW, CH, C)
    deg_parts = _sc_degree(dst_p).reshape(NC, NPAD, D)
    dinv, m1 = _tc_dinv_m1(deg_parts, feat)
    p1 = _sc_edge_pass(m1, src_p, dst_p).reshape(NC, NPAD, D)
    f1, m2 = _tc_f1_m2(feat, dinv, p1, learnable_diag)
    p2 = _sc_edge_pass(m2, src_p, dst_p).reshape(NC, NPAD, D)
    return _tc_final(feat, dinv, f1, p2, learnable_diag, W, b.reshape(1, D))


# speedup vs baseline: 3.0207x; 1.0094x over previous
import functools

import jax
import jax.numpy as jnp
from jax import lax
from jax.experimental import pallas as pl
from jax.experimental.pallas import tpu as pltpu
from jax.experimental.pallas import tpu_sc as plsc

N = 10000
E = 320000
D = 128
NC, NS = 2, 16
NW = NC * NS
C = 128
CH = 80
EPW = CH * C
EPAD = NW * EPW
DUMP = N
NPAD = 10240
STR = NPAD // NS
GRID = 10
BR = N // GRID

_mesh = plsc.VectorSubcoreMesh(core_axis_name="c", subcore_axis_name="s")


@functools.partial(
    pl.kernel,
    out_type=jax.ShapeDtypeStruct((NC * NPAD, D), jnp.float32),
    mesh=_mesh,
    scratch_types=[
        pltpu.VMEM((CH, C), jnp.int32),
        pltpu.VMEM((C, D), jnp.float32),
        pltpu.VMEM_SHARED((NPAD, D), jnp.float32),
    ],
)
def _sc_degree(dst_hbm, out_hbm, idx_v, rows_v, acc_sh):
    c = lax.axis_index("c")
    s = lax.axis_index("s")
    wid = s * NC + c
    pltpu.sync_copy(dst_hbm.at[wid], idx_v)
    zrow = jnp.zeros((16,), jnp.float32)

    def zinit(i, carry):
        for k in range(D // 16):
            rows_v[i, pl.ds(k * 16, 16)] = zrow
        return carry

    lax.fori_loop(0, C, zinit, 0)
    for k in range(STR // C):
        pltpu.sync_copy(rows_v, acc_sh.at[pl.ds(s * STR + k * C, C)])
    plsc.subcore_barrier()
    orow = jnp.ones((16,), jnp.float32)

    def oinit(i, carry):
        for k in range(D // 16):
            rows_v[i, pl.ds(k * 16, 16)] = orow
        return carry

    lax.fori_loop(0, C, oinit, 0)

    def body(j, carry):
        pltpu.sync_copy(rows_v, acc_sh.at[idx_v.at[j]], add=True)
        return carry

    lax.fori_loop(0, CH, body, 0)
    plsc.subcore_barrier()
    for k in range(STR // C):
        pltpu.sync_copy(acc_sh.at[pl.ds(s * STR + k * C, C)], rows_v)
        pltpu.sync_copy(rows_v, out_hbm.at[pl.ds(c * NPAD + s * STR + k * C, C)])


@functools.partial(
    pl.kernel,
    out_type=jax.ShapeDtypeStruct((NC * NPAD, D), jnp.float32),
    mesh=_mesh,
    scratch_types=[
        pltpu.VMEM((32, C), jnp.int32),
        pltpu.VMEM((32, C), jnp.int32),
        pltpu.VMEM((C, D), jnp.float32),
        pltpu.VMEM((C, D), jnp.float32),
        pltpu.VMEM_SHARED((NPAD, D), jnp.float32),
        pltpu.SemaphoreType.DMA,
        pltpu.SemaphoreType.DMA,
    ],
)
def _sc_edge_pass(m_hbm, src_hbm, dst_hbm, out_hbm,
                  sblk, dblk, buf_a, buf_b, acc_sh, sem_a, sem_b):
    c = lax.axis_index("c")
    s = lax.axis_index("s")
    wid = s * NC + c
    zrow = jnp.zeros((16,), jnp.float32)

    def zinit(i, carry):
        for k in range(D // 16):
            buf_a[i, pl.ds(k * 16, 16)] = zrow
        return carry

    lax.fori_loop(0, C, zinit, 0)
    for k in range(STR // C):
        pltpu.sync_copy(buf_a, acc_sh.at[pl.ds(s * STR + k * C, C)])
    plsc.subcore_barrier()

    def load_blk(b, half):
        pltpu.sync_copy(src_hbm.at[wid, pl.ds(b * 16, 16)],
                        sblk.at[pl.ds(half * 16, 16)])
        pltpu.sync_copy(dst_hbm.at[wid, pl.ds(b * 16, 16)],
                        dblk.at[pl.ds(half * 16, 16)])

    def gstart(r, buf, sem):
        pltpu.async_copy(m_hbm.at[sblk.at[r]], buf, sem)

    def gwait(buf, sem):
        pltpu.make_async_copy(m_hbm.at[sblk.at[0]], buf, sem).wait()

    load_blk(0, 0)
    gstart(0, buf_a, sem_a)

    def body(j2, carry):
        j = 2 * j2
        b_next = j // 16 + 1

        @pl.when((lax.rem(j2, 8) == 0) & (b_next < CH // 16))
        def _():
            load_blk(b_next, lax.rem(b_next, 2))

        gwait(buf_a, sem_a)
        gstart(lax.rem(j + 1, 32), buf_b, sem_b)
        pltpu.sync_copy(buf_a, acc_sh.at[dblk.at[lax.rem(j, 32)]], add=True)
        gwait(buf_b, sem_b)
        gstart(lax.rem(j + 2, 32), buf_a, sem_a)
        pltpu.sync_copy(buf_b, acc_sh.at[dblk.at[lax.rem(j + 1, 32)]], add=True)
        return carry

    lax.fori_loop(0, CH // 2, body, 0)
    gwait(buf_a, sem_a)
    plsc.subcore_barrier()
    for k in range(STR // C):
        pltpu.sync_copy(acc_sh.at[pl.ds(s * STR + k * C, C)], buf_a)
        pltpu.sync_copy(buf_a, out_hbm.at[pl.ds(c * NPAD + s * STR + k * C, C)])


def _tc_dinv_m1_body(parts_ref, feat_ref, dinv_ref, m1_ref):
    deg = parts_ref[0, :, 0:1] + parts_ref[1, :, 0:1]
    dinv = lax.rsqrt(jnp.maximum(deg, 1.0))
    dinv_ref[...] = dinv
    m1_ref[...] = feat_ref[...] * dinv


_tc_dinv_m1 = pl.pallas_call(
    _tc_dinv_m1_body,
    grid=(GRID,),
    in_specs=[
        pl.BlockSpec((NC, BR, D), lambda i: (0, i, 0)),
        pl.BlockSpec((BR, D), lambda i: (i, 0)),
    ],
    out_specs=[
        pl.BlockSpec((BR, 1), lambda i: (i, 0)),
        pl.BlockSpec((BR, D), lambda i: (i, 0)),
    ],
    out_shape=[
        jax.ShapeDtypeStruct((N, 1), jnp.float32),
        jax.ShapeDtypeStruct((N, D), jnp.float32),
    ],
)


def _tc_f1_m2_body(feat_ref, dinv_ref, parts_ref, ld_ref, f1_ref, m2_ref):
    agg = parts_ref[0] + parts_ref[1]
    dinv = dinv_ref[...]
    f1 = feat_ref[...] - (agg * dinv) * ld_ref[1:2, :]
    f1_ref[...] = f1
    m2_ref[...] = f1 * dinv


_tc_f1_m2 = pl.pallas_call(
    _tc_f1_m2_body,
    grid=(GRID,),
    in_specs=[
        pl.BlockSpec((BR, D), lambda i: (i, 0)),
        pl.BlockSpec((BR, 1), lambda i: (i, 0)),
        pl.BlockSpec((NC, BR, D), lambda i: (0, i, 0)),
        pl.BlockSpec((3, D), lambda i: (0, 0)),
    ],
    out_specs=[
        pl.BlockSpec((BR, D), lambda i: (i, 0)),
        pl.BlockSpec((BR, D), lambda i: (i, 0)),
    ],
    out_shape=[
        jax.ShapeDtypeStruct((N, D), jnp.float32),
        jax.ShapeDtypeStruct((N, D), jnp.float32),
    ],
)


def _tc_final_body(feat_ref, dinv_ref, f1_ref, parts_ref, ld_ref, w_ref, b_ref,
                   h_ref):
    agg2 = parts_ref[0] + parts_ref[1]
    dinv = dinv_ref[...]
    f1 = f1_ref[...]
    f2 = f1 - (agg2 * dinv) * ld_ref[2:3, :]
    xm = (0.5 * feat_ref[...]) * ld_ref[0:1, :]
    h_lin = lax.dot_general(
        xm, w_ref[...], (((1,), (1,)), ((), ())),
        preferred_element_type=jnp.float32,
        precision=lax.Precision.HIGHEST,
    )
    h_ref[...] = h_lin + b_ref[...] + 0.5 * f1 + 0.5 * f2


_tc_final = pl.pallas_call(
    _tc_final_body,
    grid=(GRID,),
    in_specs=[
        pl.BlockSpec((BR, D), lambda i: (i, 0)),
        pl.BlockSpec((BR, 1), lambda i: (i, 0)),
        pl.BlockSpec((BR, D), lambda i: (i, 0)),
        pl.BlockSpec((NC, BR, D), lambda i: (0, i, 0)),
        pl.BlockSpec((3, D), lambda i: (0, 0)),
        pl.BlockSpec((D, D), lambda i: (0, 0)),
        pl.BlockSpec((1, D), lambda i: (0, 0)),
    ],
    out_specs=pl.BlockSpec((BR, D), lambda i: (i, 0)),
    out_shape=jax.ShapeDtypeStruct((N, D), jnp.float32),
)


def kernel(feat, edge_index, learnable_diag, W, b):
    src = edge_index[0].astype(jnp.int32)
    dst = edge_index[1].astype(jnp.int32)
    pad = EPAD - E
    src_p = jnp.concatenate([src, jnp.zeros((pad,), jnp.int32)]).reshape(NW, CH, C)
    dst_p = jnp.concatenate([dst, jnp.full((pad,), DUMP, jnp.int32)]).reshape(NW, CH, C)
    deg_parts = _sc_degree(dst_p).reshape(NC, NPAD, D)
    dinv, m1 = _tc_dinv_m1(deg_parts, feat)
    p1 = _sc_edge_pass(m1, src_p, dst_p).reshape(NC, NPAD, D)
    f1, m2 = _tc_f1_m2(feat, dinv, p1, learnable_diag)
    p2 = _sc_edge_pass(m2, src_p, dst_p).reshape(NC, NPAD, D)
    return _tc_final(feat, dinv, f1, p2, learnable_diag, W, b.reshape(1, D))


# speedup vs baseline: 3.4988x; 1.1583x over previous
import functools

import jax
import jax.numpy as jnp
from jax import lax
from jax.experimental import pallas as pl
from jax.experimental.pallas import tpu as pltpu
from jax.experimental.pallas import tpu_sc as plsc

N = 10000
E = 320000
D = 128
NC, NS = 2, 16
NW = NC * NS
C = 128
CH = 80
CH0 = 128
CH1 = 32
EPW = CH * C
EPAD = NW * EPW
DUMP = N
NPAD = 10240
STR = NPAD // NS
GRID = 10
BR = N // GRID

_mesh = plsc.VectorSubcoreMesh(core_axis_name="c", subcore_axis_name="s")


@functools.partial(
    pl.kernel,
    out_type=jax.ShapeDtypeStruct((NC * NPAD, D), jnp.float32),
    mesh=_mesh,
    scratch_types=[
        pltpu.VMEM((CH, C), jnp.int32),
        pltpu.VMEM((C, D), jnp.float32),
        pltpu.VMEM_SHARED((NPAD, D), jnp.float32),
    ],
)
def _sc_degree(dst_hbm, out_hbm, idx_v, rows_v, acc_sh):
    c = lax.axis_index("c")
    s = lax.axis_index("s")
    wid = s * NC + c
    pltpu.sync_copy(dst_hbm.at[wid], idx_v)
    zrow = jnp.zeros((16,), jnp.float32)

    def zinit(i, carry):
        for k in range(D // 16):
            rows_v[i, pl.ds(k * 16, 16)] = zrow
        return carry

    lax.fori_loop(0, C, zinit, 0)
    for k in range(STR // C):
        pltpu.sync_copy(rows_v, acc_sh.at[pl.ds(s * STR + k * C, C)])
    plsc.subcore_barrier()
    orow = jnp.ones((16,), jnp.float32)

    def oinit(i, carry):
        for k in range(D // 16):
            rows_v[i, pl.ds(k * 16, 16)] = orow
        return carry

    lax.fori_loop(0, C, oinit, 0)

    def body(j, carry):
        pltpu.sync_copy(rows_v, acc_sh.at[idx_v.at[j]], add=True)
        return carry

    lax.fori_loop(0, CH, body, 0)
    plsc.subcore_barrier()
    for k in range(STR // C):
        pltpu.sync_copy(acc_sh.at[pl.ds(s * STR + k * C, C)], rows_v)
        pltpu.sync_copy(rows_v, out_hbm.at[pl.ds(c * NPAD + s * STR + k * C, C)])


@functools.partial(
    pl.kernel,
    out_type=jax.ShapeDtypeStruct((NC * NPAD, D), jnp.float32),
    mesh=_mesh,
    scratch_types=[
        pltpu.VMEM((32, C), jnp.int32),
        pltpu.VMEM((32, C), jnp.int32),
        pltpu.VMEM((C, D), jnp.float32),
        pltpu.VMEM((C, D), jnp.float32),
        pltpu.VMEM_SHARED((NPAD, D), jnp.float32),
        pltpu.SemaphoreType.DMA,
        pltpu.SemaphoreType.DMA,
    ],
)
def _sc_edge_pass(m_hbm, src_hbm, dst_hbm, out_hbm,
                  sblk, dblk, buf_a, buf_b, acc_sh, sem_a, sem_b):
    c = lax.axis_index("c")
    s = lax.axis_index("s")
    base = lax.select(c == 0, 0, CH0)
    nch = lax.select(c == 0, CH0, CH1)
    zrow = jnp.zeros((16,), jnp.float32)

    def zinit(i, carry):
        for k in range(D // 16):
            buf_a[i, pl.ds(k * 16, 16)] = zrow
        return carry

    lax.fori_loop(0, C, zinit, 0)
    for k in range(STR // C):
        pltpu.sync_copy(buf_a, acc_sh.at[pl.ds(s * STR + k * C, C)])
    plsc.subcore_barrier()

    def load_blk(b, half):
        pltpu.sync_copy(src_hbm.at[s, pl.ds(base + b * 16, 16)],
                        sblk.at[pl.ds(half * 16, 16)])
        pltpu.sync_copy(dst_hbm.at[s, pl.ds(base + b * 16, 16)],
                        dblk.at[pl.ds(half * 16, 16)])

    def gstart(r, buf, sem):
        pltpu.async_copy(m_hbm.at[sblk.at[r]], buf, sem)

    def gwait(buf, sem):
        pltpu.make_async_copy(m_hbm.at[sblk.at[0]], buf, sem).wait()

    load_blk(0, 0)
    gstart(0, buf_a, sem_a)

    def body(j2, carry):
        j = 2 * j2
        b_next = j // 16 + 1

        @pl.when((lax.rem(j2, 8) == 0) & (b_next < nch // 16))
        def _():
            load_blk(b_next, lax.rem(b_next, 2))

        gwait(buf_a, sem_a)
        gstart(lax.rem(j + 1, 32), buf_b, sem_b)
        pltpu.sync_copy(buf_a, acc_sh.at[dblk.at[lax.rem(j, 32)]], add=True)
        gwait(buf_b, sem_b)
        gstart(lax.rem(j + 2, 32), buf_a, sem_a)
        pltpu.sync_copy(buf_b, acc_sh.at[dblk.at[lax.rem(j + 1, 32)]], add=True)
        return carry

    lax.fori_loop(0, nch // 2, body, 0)
    gwait(buf_a, sem_a)
    plsc.subcore_barrier()
    for k in range(STR // C):
        pltpu.sync_copy(acc_sh.at[pl.ds(s * STR + k * C, C)], buf_a)
        pltpu.sync_copy(buf_a, out_hbm.at[pl.ds(c * NPAD + s * STR + k * C, C)])


def _tc_dinv_m1_body(parts_ref, feat_ref, dinv_ref, m1_ref):
    deg = parts_ref[0, :, 0:1] + parts_ref[1, :, 0:1]
    dinv = lax.rsqrt(jnp.maximum(deg, 1.0))
    dinv_ref[...] = dinv
    m1_ref[...] = feat_ref[...] * dinv


_tc_dinv_m1 = pl.pallas_call(
    _tc_dinv_m1_body,
    grid=(GRID,),
    in_specs=[
        pl.BlockSpec((NC, BR, D), lambda i: (0, i, 0)),
        pl.BlockSpec((BR, D), lambda i: (i, 0)),
    ],
    out_specs=[
        pl.BlockSpec((BR, 1), lambda i: (i, 0)),
        pl.BlockSpec((BR, D), lambda i: (i, 0)),
    ],
    out_shape=[
        jax.ShapeDtypeStruct((N, 1), jnp.float32),
        jax.ShapeDtypeStruct((N, D), jnp.float32),
    ],
)


def _tc_f1_m2_body(feat_ref, dinv_ref, parts_ref, ld_ref, f1_ref, m2_ref):
    agg = parts_ref[0] + parts_ref[1]
    dinv = dinv_ref[...]
    f1 = feat_ref[...] - (agg * dinv) * ld_ref[1:2, :]
    f1_ref[...] = f1
    m2_ref[...] = f1 * dinv


_tc_f1_m2 = pl.pallas_call(
    _tc_f1_m2_body,
    grid=(GRID,),
    in_specs=[
        pl.BlockSpec((BR, D), lambda i: (i, 0)),
        pl.BlockSpec((BR, 1), lambda i: (i, 0)),
        pl.BlockSpec((NC, BR, D), lambda i: (0, i, 0)),
        pl.BlockSpec((3, D), lambda i: (0, 0)),
    ],
    out_specs=[
        pl.BlockSpec((BR, D), lambda i: (i, 0)),
        pl.BlockSpec((BR, D), lambda i: (i, 0)),
    ],
    out_shape=[
        jax.ShapeDtypeStruct((N, D), jnp.float32),
        jax.ShapeDtypeStruct((N, D), jnp.float32),
    ],
)


def _tc_final_body(feat_ref, dinv_ref, f1_ref, parts_ref, ld_ref, w_ref, b_ref,
                   h_ref):
    agg2 = parts_ref[0] + parts_ref[1]
    dinv = dinv_ref[...]
    f1 = f1_ref[...]
    f2 = f1 - (agg2 * dinv) * ld_ref[2:3, :]
    xm = (0.5 * feat_ref[...]) * ld_ref[0:1, :]
    h_lin = lax.dot_general(
        xm, w_ref[...], (((1,), (1,)), ((), ())),
        preferred_element_type=jnp.float32,
        precision=lax.Precision.HIGHEST,
    )
    h_ref[...] = h_lin + b_ref[...] + 0.5 * f1 + 0.5 * f2


_tc_final = pl.pallas_call(
    _tc_final_body,
    grid=(GRID,),
    in_specs=[
        pl.BlockSpec((BR, D), lambda i: (i, 0)),
        pl.BlockSpec((BR, 1), lambda i: (i, 0)),
        pl.BlockSpec((BR, D), lambda i: (i, 0)),
        pl.BlockSpec((NC, BR, D), lambda i: (0, i, 0)),
        pl.BlockSpec((3, D), lambda i: (0, 0)),
        pl.BlockSpec((D, D), lambda i: (0, 0)),
        pl.BlockSpec((1, D), lambda i: (0, 0)),
    ],
    out_specs=pl.BlockSpec((BR, D), lambda i: (i, 0)),
    out_shape=jax.ShapeDtypeStruct((N, D), jnp.float32),
)


def kernel(feat, edge_index, learnable_diag, W, b):
    src = edge_index[0].astype(jnp.int32)
    dst = edge_index[1].astype(jnp.int32)
    pad = EPAD - E
    src_flat = jnp.concatenate([src, jnp.zeros((pad,), jnp.int32)])
    dst_flat = jnp.concatenate([dst, jnp.full((pad,), DUMP, jnp.int32)])
    src_e = src_flat.reshape(NS, CH0 + CH1, C)
    dst_e = dst_flat.reshape(NS, CH0 + CH1, C)
    dst_p = dst_flat.reshape(NW, CH, C)
    deg_parts = _sc_degree(dst_p).reshape(NC, NPAD, D)
    dinv, m1 = _tc_dinv_m1(deg_parts, feat)
    p1 = _sc_edge_pass(m1, src_e, dst_e).reshape(NC, NPAD, D)
    f1, m2 = _tc_f1_m2(feat, dinv, p1, learnable_diag)
    p2 = _sc_edge_pass(m2, src_e, dst_e).reshape(NC, NPAD, D)
    return _tc_final(feat, dinv, f1, p2, learnable_diag, W, b.reshape(1, D))


# speedup vs baseline: 8.5014x; 2.4298x over previous
import functools

import jax
import jax.numpy as jnp
from jax import lax
from jax.experimental import pallas as pl
from jax.experimental.pallas import tpu as pltpu
from jax.experimental.pallas import tpu_sc as plsc

N = 10000
E = 320000
D = 128
NC, NS = 2, 16
NW = NC * NS
C = 128
CH = 80
CH0 = 80
CH1 = 80
EPW = CH * C
EPAD = NW * EPW
DUMP = N
NPAD = 10240
STR = NPAD // NS
GRID = 10
BR = N // GRID

_mesh = plsc.VectorSubcoreMesh(core_axis_name="c", subcore_axis_name="s")


@functools.partial(
    pl.kernel,
    out_type=jax.ShapeDtypeStruct((NC * NPAD, D), jnp.float32),
    mesh=_mesh,
    scratch_types=[
        pltpu.VMEM((CH, C), jnp.int32),
        pltpu.VMEM((C, D), jnp.float32),
        pltpu.VMEM_SHARED((NPAD, D), jnp.float32),
    ],
)
def _sc_degree(dst_hbm, out_hbm, idx_v, rows_v, acc_sh):
    c = lax.axis_index("c")
    s = lax.axis_index("s")
    wid = s * NC + c
    pltpu.sync_copy(dst_hbm.at[wid], idx_v)
    zrow = jnp.zeros((16,), jnp.float32)

    def zinit(i, carry):
        for k in range(D // 16):
            rows_v[i, pl.ds(k * 16, 16)] = zrow
        return carry

    lax.fori_loop(0, C, zinit, 0)
    for k in range(STR // C):
        pltpu.sync_copy(rows_v, acc_sh.at[pl.ds(s * STR + k * C, C)])
    plsc.subcore_barrier()
    orow = jnp.ones((16,), jnp.float32)

    def oinit(i, carry):
        for k in range(D // 16):
            rows_v[i, pl.ds(k * 16, 16)] = orow
        return carry

    lax.fori_loop(0, C, oinit, 0)

    def body(j, carry):
        pltpu.sync_copy(rows_v, acc_sh.at[idx_v.at[j]], add=True)
        return carry

    lax.fori_loop(0, CH, body, 0)
    plsc.subcore_barrier()
    for k in range(STR // C):
        pltpu.sync_copy(acc_sh.at[pl.ds(s * STR + k * C, C)], rows_v)
        pltpu.sync_copy(rows_v, out_hbm.at[pl.ds(c * NPAD + s * STR + k * C, C)])


@functools.partial(
    pl.kernel,
    out_type=jax.ShapeDtypeStruct((NC * NPAD, D), jnp.float32),
    mesh=_mesh,
    scratch_types=[
        pltpu.VMEM((32, C), jnp.int32),
        pltpu.VMEM((32, C), jnp.int32),
        pltpu.VMEM((C, D), jnp.float32),
        pltpu.VMEM((C, D), jnp.float32),
        pltpu.VMEM_SHARED((NPAD, D), jnp.float32),
        pltpu.SemaphoreType.DMA,
        pltpu.SemaphoreType.DMA,
    ],
)
def _sc_edge_pass(m_hbm, src_hbm, dst_hbm, out_hbm,
                  sblk, dblk, buf_a, buf_b, acc_sh, sem_a, sem_b):
    c = lax.axis_index("c")
    s = lax.axis_index("s")
    base = lax.select(c == 0, 0, CH0)
    nch = lax.select(c == 0, CH0, CH1)
    zrow = jnp.zeros((16,), jnp.float32)

    def zinit(i, carry):
        for k in range(D // 16):
            buf_a[i, pl.ds(k * 16, 16)] = zrow
        return carry

    lax.fori_loop(0, C, zinit, 0)
    for k in range(STR // C):
        pltpu.sync_copy(buf_a, acc_sh.at[pl.ds(s * STR + k * C, C)])
    plsc.subcore_barrier()

    def load_blk(b, half):
        pltpu.sync_copy(src_hbm.at[s, pl.ds(base + b * 16, 16)],
                        sblk.at[pl.ds(half * 16, 16)])
        pltpu.sync_copy(dst_hbm.at[s, pl.ds(base + b * 16, 16)],
                        dblk.at[pl.ds(half * 16, 16)])

    def gstart(r, buf, sem):
        pltpu.async_copy(m_hbm.at[sblk.at[r]], buf, sem)

    def gwait(buf, sem):
        pltpu.make_async_copy(m_hbm.at[sblk.at[0]], buf, sem).wait()

    load_blk(0, 0)
    gstart(0, buf_a, sem_a)

    def body(j2, carry):
        j = 2 * j2
        b_next = j // 16 + 1

        @pl.when((lax.rem(j2, 8) == 0) & (b_next < nch // 16))
        def _():
            load_blk(b_next, lax.rem(b_next, 2))

        gwait(buf_a, sem_a)
        gstart(lax.rem(j + 1, 32), buf_b, sem_b)
        pltpu.sync_copy(buf_a, acc_sh.at[dblk.at[lax.rem(j, 32)]], add=True)
        gwait(buf_b, sem_b)
        gstart(lax.rem(j + 2, 32), buf_a, sem_a)
        pltpu.sync_copy(buf_b, acc_sh.at[dblk.at[lax.rem(j + 1, 32)]], add=True)
        return carry

    lax.fori_loop(0, nch // 2, body, 0)
    gwait(buf_a, sem_a)
    plsc.subcore_barrier()
    for k in range(STR // C):
        pltpu.sync_copy(acc_sh.at[pl.ds(s * STR + k * C, C)], buf_a)
        pltpu.sync_copy(buf_a, out_hbm.at[pl.ds(c * NPAD + s * STR + k * C, C)])


def _tc_dinv_m1_body(parts_ref, feat_ref, dinv_ref, m1_ref):
    deg = parts_ref[0, :, 0:1] + parts_ref[1, :, 0:1]
    dinv = lax.rsqrt(jnp.maximum(deg, 1.0))
    dinv_ref[...] = dinv
    m1_ref[...] = feat_ref[...] * dinv


_tc_dinv_m1 = pl.pallas_call(
    _tc_dinv_m1_body,
    grid=(GRID,),
    in_specs=[
        pl.BlockSpec((NC, BR, D), lambda i: (0, i, 0)),
        pl.BlockSpec((BR, D), lambda i: (i, 0)),
    ],
    out_specs=[
        pl.BlockSpec((BR, 1), lambda i: (i, 0)),
        pl.BlockSpec((BR, D), lambda i: (i, 0)),
    ],
    out_shape=[
        jax.ShapeDtypeStruct((N, 1), jnp.float32),
        jax.ShapeDtypeStruct((N, D), jnp.float32),
    ],
)


def _tc_f1_m2_body(feat_ref, dinv_ref, parts_ref, ld_ref, f1_ref, m2_ref):
    agg = parts_ref[0] + parts_ref[1]
    dinv = dinv_ref[...]
    f1 = feat_ref[...] - (agg * dinv) * ld_ref[1:2, :]
    f1_ref[...] = f1
    m2_ref[...] = f1 * dinv


_tc_f1_m2 = pl.pallas_call(
    _tc_f1_m2_body,
    grid=(GRID,),
    in_specs=[
        pl.BlockSpec((BR, D), lambda i: (i, 0)),
        pl.BlockSpec((BR, 1), lambda i: (i, 0)),
        pl.BlockSpec((NC, BR, D), lambda i: (0, i, 0)),
        pl.BlockSpec((3, D), lambda i: (0, 0)),
    ],
    out_specs=[
        pl.BlockSpec((BR, D), lambda i: (i, 0)),
        pl.BlockSpec((BR, D), lambda i: (i, 0)),
    ],
    out_shape=[
        jax.ShapeDtypeStruct((N, D), jnp.float32),
        jax.ShapeDtypeStruct((N, D), jnp.float32),
    ],
)


def _tc_final_body(feat_ref, dinv_ref, f1_ref, parts_ref, ld_ref, w_ref, b_ref,
                   h_ref):
    agg2 = parts_ref[0] + parts_ref[1]
    dinv = dinv_ref[...]
    f1 = f1_ref[...]
    f2 = f1 - (agg2 * dinv) * ld_ref[2:3, :]
    xm = (0.5 * feat_ref[...]) * ld_ref[0:1, :]
    h_lin = lax.dot_general(
        xm, w_ref[...], (((1,), (1,)), ((), ())),
        preferred_element_type=jnp.float32,
        precision=lax.Precision.HIGHEST,
    )
    h_ref[...] = h_lin + b_ref[...] + 0.5 * f1 + 0.5 * f2


_tc_final = pl.pallas_call(
    _tc_final_body,
    grid=(GRID,),
    in_specs=[
        pl.BlockSpec((BR, D), lambda i: (i, 0)),
        pl.BlockSpec((BR, 1), lambda i: (i, 0)),
        pl.BlockSpec((BR, D), lambda i: (i, 0)),
        pl.BlockSpec((NC, BR, D), lambda i: (0, i, 0)),
        pl.BlockSpec((3, D), lambda i: (0, 0)),
        pl.BlockSpec((D, D), lambda i: (0, 0)),
        pl.BlockSpec((1, D), lambda i: (0, 0)),
    ],
    out_specs=pl.BlockSpec((BR, D), lambda i: (i, 0)),
    out_shape=jax.ShapeDtypeStruct((N, D), jnp.float32),
)


def kernel(feat, edge_index, learnable_diag, W, b):
    src = edge_index[0].astype(jnp.int32)
    dst = edge_index[1].astype(jnp.int32)
    pad = EPAD - E
    pad_src = (jnp.arange(pad, dtype=jnp.int32) * 131) % N
    src_flat = jnp.concatenate([src, pad_src])
    dst_flat = jnp.concatenate([dst, jnp.full((pad,), DUMP, jnp.int32)])
    src_e = src_flat.reshape(NS, CH0 + CH1, C)
    dst_e = dst_flat.reshape(NS, CH0 + CH1, C)
    dst_p = dst_flat.reshape(NW, CH, C)
    deg_parts = _sc_degree(dst_p).reshape(NC, NPAD, D)
    dinv, m1 = _tc_dinv_m1(deg_parts, feat)
    p1 = _sc_edge_pass(m1, src_e, dst_e).reshape(NC, NPAD, D)
    f1, m2 = _tc_f1_m2(feat, dinv, p1, learnable_diag)
    p2 = _sc_edge_pass(m2, src_e, dst_e).reshape(NC, NPAD, D)
    return _tc_final(feat, dinv, f1, p2, learnable_diag, W, b.reshape(1, D))
